# baseline (speedup 1.0000x reference)
import functools

import jax
import jax.numpy as jnp
from jax import lax
from jax.experimental import pallas as pl
from jax.experimental.pallas import tpu as pltpu
from jax.experimental.pallas import tpu_sc as plsc

_DEPTH = 1000
_ROWS = 4096 * 26
_NW = 32
_RPW = _ROWS // _NW
_ZR = 104
_NZ = _RPW // _ZR
_NIDX = _RPW // 128


def _make_kernel():
    mesh = plsc.VectorSubcoreMesh(core_axis_name="c", subcore_axis_name="s")

    @functools.partial(
        pl.kernel,
        mesh=mesh,
        out_type=jax.ShapeDtypeStruct((_ROWS * _DEPTH,), jnp.float32),
        scratch_types=[
            pltpu.VMEM((_RPW,), jnp.int32),
            pltpu.VMEM((_ZR * _DEPTH,), jnp.float32),
            pltpu.VMEM((_NIDX, 128), jnp.int32),
            pltpu.VMEM((128,), jnp.float32),
            pltpu.SemaphoreType.DMA,
            pltpu.SemaphoreType.DMA,
        ],
        compiler_params=pltpu.CompilerParams(needs_layout_passes=False),
    )
    def onehot(idx_hbm, out_hbm, idx_v, zbuf, pos_v, ones_v, zsem, ssem):
        wid = lax.axis_index("s") * 2 + lax.axis_index("c")
        base_row = wid * _RPW
        pltpu.sync_copy(idx_hbm.at[pl.ds(base_row, _RPW)], idx_v)

        zeros = jnp.zeros((16,), jnp.float32)
        ones = jnp.ones((16,), jnp.float32)
        lane = lax.iota(jnp.int32, 16)

        def zbody(j, carry):
            zbuf[pl.ds(j * 16, 16)] = zeros
            return carry

        lax.fori_loop(0, _ZR * _DEPTH // 16, zbody, 0)
        for g in range(128 // 16):
            ones_v[pl.ds(g * 16, 16)] = ones

        def zfire(m, carry):
            dst = out_hbm.at[pl.ds((base_row + m * _ZR) * _DEPTH, _ZR * _DEPTH)]
            pltpu.async_copy(zbuf, dst, zsem)
            return carry

        lax.fori_loop(0, _NZ, zfire, 0)

        def pbody(j, carry):
            for g in range(128 // 16):
                i = j * 128 + g * 16
                idxv = idx_v[pl.ds(i, 16)]
                pos_v[j, pl.ds(g * 16, 16)] = (base_row + i + lane) * _DEPTH + idxv
            return carry

        lax.fori_loop(0, _NIDX, pbody, 0)

        def zdrain(m, carry):
            dst = out_hbm.at[pl.ds((base_row + m * _ZR) * _DEPTH, _ZR * _DEPTH)]
            pltpu.make_async_copy(zbuf, dst, zsem).wait()
            return carry

        lax.fori_loop(0, _NZ, zdrain, 0)

        def sfire(j, carry):
            pltpu.async_copy(ones_v, out_hbm.at[pos_v.at[j]], ssem)
            return carry

        lax.fori_loop(0, _NIDX, sfire, 0)

        def sdrain(j, carry):
            pltpu.make_async_copy(ones_v, out_hbm.at[pos_v.at[j]], ssem).wait()
            return carry

        lax.fori_loop(0, _NIDX, sdrain, 0)

    return onehot


_onehot = _make_kernel()


def kernel(inputs):
    idx = jnp.asarray(inputs, jnp.int32).reshape(-1)
    flat = _onehot(idx)
    return flat.reshape(inputs.shape[0], inputs.shape[1], _DEPTH)


# speedup vs baseline: 1.0099x; 1.0099x over previous
import functools

import jax
import jax.numpy as jnp
from jax import lax
from jax.experimental import pallas as pl
from jax.experimental.pallas import tpu as pltpu
from jax.experimental.pallas import tpu_sc as plsc

_DEPTH = 1000
_ROWS = 4096 * 26
_NW = 32
_RPW = _ROWS // _NW
_WIN = 832
_NZ = _RPW // _WIN
_ZSLICE = _WIN * _DEPTH // 16
_NIDX = _RPW // 128


def _make_kernel():
    mesh = plsc.VectorSubcoreMesh(core_axis_name="c", subcore_axis_name="s")

    @functools.partial(
        pl.kernel,
        mesh=mesh,
        out_type=jax.ShapeDtypeStruct((_ROWS * _DEPTH,), jnp.float32),
        scratch_types=[
            pltpu.VMEM((_RPW,), jnp.int32),
            pltpu.VMEM((_ZSLICE,), jnp.float32),
            pltpu.VMEM((_NIDX, 128), jnp.int32),
            pltpu.VMEM((_NIDX, 128), jnp.float32),
            pltpu.VMEM_SHARED((_WIN * _DEPTH,), jnp.float32),
            pltpu.SemaphoreType.DMA,
            pltpu.SemaphoreType.DMA,
        ],
        compiler_params=pltpu.CompilerParams(needs_layout_passes=False),
    )
    def onehot(idx_hbm, out_hbm, idx_v, zbuf, pos_v, ones_v, zshared, zsem, ssem):
        cid = lax.axis_index("c")
        sid = lax.axis_index("s")
        wid = sid * 2 + cid
        base_row = wid * _RPW
        pltpu.sync_copy(idx_hbm.at[pl.ds(base_row, _RPW)], idx_v)

        zeros = jnp.zeros((16,), jnp.float32)
        ones = jnp.ones((16,), jnp.float32)
        lane = lax.iota(jnp.int32, 16)

        def zbody(j, carry):
            zbuf[pl.ds(j * 16, 16)] = zeros
            return carry

        lax.fori_loop(0, _ZSLICE // 16, zbody, 0)
        for j in range(_NIDX):
            for g in range(128 // 16):
                ones_v[j, pl.ds(g * 16, 16)] = ones

        pltpu.sync_copy(zbuf, zshared.at[pl.ds(sid * _ZSLICE, _ZSLICE)])
        plsc.subcore_barrier()

        def zfire(m, carry):
            dst = out_hbm.at[pl.ds((base_row + m * _WIN) * _DEPTH, _WIN * _DEPTH)]
            pltpu.async_copy(zshared, dst, zsem)
            return carry

        lax.fori_loop(0, _NZ, zfire, 0)

        def pbody(j, carry):
            for g in range(128 // 16):
                i = j * 128 + g * 16
                idxv = idx_v[pl.ds(i, 16)]
                pos_v[j, pl.ds(g * 16, 16)] = (base_row + i + lane) * _DEPTH + idxv
            return carry

        lax.fori_loop(0, _NIDX, pbody, 0)

        def zdrain(m, carry):
            dst = out_hbm.at[pl.ds((base_row + m * _WIN) * _DEPTH, _WIN * _DEPTH)]
            pltpu.make_async_copy(zshared, dst, zsem).wait()
            return carry

        lax.fori_loop(0, _NZ, zdrain, 0)

        def sfire(j, carry):
            pltpu.async_copy(ones_v.at[j], out_hbm.at[pos_v.at[j]], ssem)
            return carry

        lax.fori_loop(0, _NIDX, sfire, 0)

        def sdrain(j, carry):
            pltpu.make_async_copy(ones_v.at[j], out_hbm.at[pos_v.at[j]], ssem).wait()
            return carry

        lax.fori_loop(0, _NIDX, sdrain, 0)

    return onehot


_onehot = _make_kernel()


def kernel(inputs):
    idx = jnp.asarray(inputs, jnp.int32).reshape(-1)
    flat = _onehot(idx)
    return flat.reshape(inputs.shape[0], inputs.shape[1], _DEPTH)
